# baseline (speedup 1.0000x reference)
import jax
import jax.numpy as jnp
from jax import lax
from jax.experimental import pallas as pl
from jax.experimental.pallas import tpu as pltpu
from jax.experimental.pallas import tpu_sc as plsc

B, C, H, W = 8, 3, 512, 512
MAX_DISP = 32.0
NC, NS = 2, 16
NW = NC * NS
W_PER_B = NW // B
ROWS_PER_W = H // W_PER_B
RBLK = 16
NBLK = ROWS_PER_W // RBLK
NCH = W // 16
F2P23 = 8388608.0


def _body(image_hbm, depth_hbm, out_l_hbm, out_r_hbm,
          depth_v, img_v, cnt_v, acc_v):
    wid = lax.axis_index("s") * NC + lax.axis_index("c")
    b = wid // W_PER_B
    y_base = (wid % W_PER_B) * ROWS_PER_W

    xiota = lax.iota(jnp.int32, 16)
    ones = jnp.ones((16,), jnp.float32)
    zeros = jnp.zeros((16,), jnp.float32)
    dvecs = [jnp.full((16,), di, jnp.int32) for di in range(2)]
    cvecs = [jnp.full((16,), c, jnp.int32) for c in range(C)]

    def do_block(blk, carry):
        ys = y_base + blk * RBLK
        pltpu.sync_copy(depth_hbm.at[b, pl.ds(ys, RBLK), :], depth_v)
        pltpu.sync_copy(image_hbm.at[b, :, pl.ds(ys, RBLK), :], img_v)

        @plsc.parallel_loop(0, RBLK * NCH, 1, unroll=4)
        def zero_k(k):
            r = k // NCH
            xo = (k % NCH) * 16
            for di in range(2):
                cnt_v[di, r, pl.ds(xo, 16)] = zeros
                for c in range(C):
                    acc_v[di, c, r, pl.ds(xo, 16)] = zeros

        @plsc.parallel_loop(0, RBLK * NCH, 1, unroll=4)
        def scat_k(k):
            r = k // NCH
            xo = (k % NCH) * 16
            d16 = depth_v[r, pl.ds(xo, 16)]
            disp = ((d16 * MAX_DISP + F2P23) - F2P23).astype(jnp.int32)
            xb = xiota + xo
            rr = jnp.broadcast_to(r, (16,))
            vals = [img_v[c, r, pl.ds(xo, 16)] for c in range(C)]
            for di in range(2):
                xt = xb - disp if di == 0 else xb + disp
                msk = (xt >= 0) & (xt < W)
                xtc = jnp.clip(xt, 0, W - 1)
                plsc.addupdate_scatter(cnt_v, [dvecs[di], rr, xtc],
                                       ones, mask=msk)
                for c in range(C):
                    plsc.addupdate_scatter(acc_v, [dvecs[di], cvecs[c], rr, xtc],
                                           vals[c], mask=msk)

        @plsc.parallel_loop(0, RBLK * NCH, 1, unroll=4)
        def fin_k(k):
            r = k // NCH
            xo = (k % NCH) * 16
            for di in range(2):
                cnt = cnt_v[di, r, pl.ds(xo, 16)]
                inv = 1.0 / jnp.maximum(cnt, 1.0)
                for c in range(C):
                    a = acc_v[di, c, r, pl.ds(xo, 16)]
                    acc_v[di, c, r, pl.ds(xo, 16)] = jnp.clip(a * inv, 0.0, 1.0)

        pltpu.sync_copy(acc_v.at[0], out_l_hbm.at[b, :, pl.ds(ys, RBLK), :])
        pltpu.sync_copy(acc_v.at[1], out_r_hbm.at[b, :, pl.ds(ys, RBLK), :])
        return carry

    lax.fori_loop(0, NBLK, do_block, 0)


def kernel(image, depth):
    mesh = plsc.VectorSubcoreMesh(core_axis_name="c", subcore_axis_name="s",
                                  num_cores=NC, num_subcores=NS)
    f = pl.kernel(
        _body,
        out_type=(jax.ShapeDtypeStruct((B, C, H, W), jnp.float32),
                  jax.ShapeDtypeStruct((B, C, H, W), jnp.float32)),
        mesh=mesh,
        scratch_types=[
            pltpu.VMEM((RBLK, W), jnp.float32),
            pltpu.VMEM((C, RBLK, W), jnp.float32),
            pltpu.VMEM((2, RBLK, W), jnp.float32),
            pltpu.VMEM((2, C, RBLK, W), jnp.float32),
        ],
        compiler_params=pltpu.CompilerParams(use_tc_tiling_on_sc=False,
                                             needs_layout_passes=False),
    )
    return f(image, depth)


# speedup vs baseline: 1.5941x; 1.5941x over previous
import jax
import jax.numpy as jnp
from jax import lax
from jax.experimental import pallas as pl
from jax.experimental.pallas import tpu as pltpu
from jax.experimental.pallas import tpu_sc as plsc

B, C, H, W = 8, 3, 512, 512
MAX_DISP = 32.0
NC, NS = 2, 16
NW = NC * NS
W_PER_B = NW // B
ROWS_PER_W = H // W_PER_B
RBLK = 16
NBLK = ROWS_PER_W // RBLK
NCH = W // 16
F2P23 = 8388608.0


def _body(image_hbm, depth_hbm, out_l_hbm, out_r_hbm,
          depth_v, img_v, cnt_v, acc_v):
    wid = lax.axis_index("s") * NC + lax.axis_index("c")
    b = wid // W_PER_B
    y_base = (wid % W_PER_B) * ROWS_PER_W

    xiota = lax.iota(jnp.int32, 16)
    ones = jnp.ones((16,), jnp.float32)
    zeros = jnp.zeros((16,), jnp.float32)
    dvecs = [jnp.full((16,), di, jnp.int32) for di in range(2)]
    cvecs = [jnp.full((16,), c, jnp.int32) for c in range(C)]

    def do_block(blk, carry):
        ys = y_base + blk * RBLK
        pltpu.sync_copy(depth_hbm.at[b, pl.ds(ys, RBLK), :], depth_v)
        pltpu.sync_copy(image_hbm.at[b, :, pl.ds(ys, RBLK), :], img_v)

        @plsc.parallel_loop(0, RBLK * NCH, 1, unroll=2)
        def zero_k(k):
            r = k // NCH
            xo = (k % NCH) * 16
            for di in range(2):
                cnt_v[di, r, pl.ds(xo, 16)] = zeros
                for c in range(C):
                    acc_v[di, c, r, pl.ds(xo, 16)] = zeros

        @plsc.parallel_loop(0, RBLK * NCH, 1, unroll=2)
        def scat_k(k):
            r = k // NCH
            xo = (k % NCH) * 16
            d16 = depth_v[r, pl.ds(xo, 16)]
            disp = ((d16 * MAX_DISP + F2P23) - F2P23).astype(jnp.int32)
            xb = xiota + xo
            rr = jnp.broadcast_to(r, (16,))
            vals = [img_v[c, r, pl.ds(xo, 16)] for c in range(C)]
            for di in range(2):
                xt = xb - disp if di == 0 else xb + disp
                msk = (xt >= 0) & (xt < W)
                xtc = jnp.clip(xt, 0, W - 1)
                plsc.addupdate_scatter(cnt_v, [dvecs[di], rr, xtc],
                                       ones, mask=msk)
                for c in range(C):
                    plsc.addupdate_scatter(acc_v, [dvecs[di], cvecs[c], rr, xtc],
                                           vals[c], mask=msk)

        @plsc.parallel_loop(0, RBLK * NCH, 1, unroll=2)
        def fin_k(k):
            r = k // NCH
            xo = (k % NCH) * 16
            for di in range(2):
                cnt = cnt_v[di, r, pl.ds(xo, 16)]
                inv = 1.0 / jnp.maximum(cnt, 1.0)
                for c in range(C):
                    a = acc_v[di, c, r, pl.ds(xo, 16)]
                    acc_v[di, c, r, pl.ds(xo, 16)] = jnp.clip(a * inv, 0.0, 1.0)

        pltpu.sync_copy(acc_v.at[0], out_l_hbm.at[b, :, pl.ds(ys, RBLK), :])
        pltpu.sync_copy(acc_v.at[1], out_r_hbm.at[b, :, pl.ds(ys, RBLK), :])
        return carry

    lax.fori_loop(0, NBLK, do_block, 0)


def kernel(image, depth):
    mesh = plsc.VectorSubcoreMesh(core_axis_name="c", subcore_axis_name="s",
                                  num_cores=NC, num_subcores=NS)
    f = pl.kernel(
        _body,
        out_type=(jax.ShapeDtypeStruct((B, C, H, W), jnp.float32),
                  jax.ShapeDtypeStruct((B, C, H, W), jnp.float32)),
        mesh=mesh,
        scratch_types=[
            pltpu.VMEM((RBLK, W), jnp.float32),
            pltpu.VMEM((C, RBLK, W), jnp.float32),
            pltpu.VMEM((2, RBLK, W), jnp.float32),
            pltpu.VMEM((2, C, RBLK, W), jnp.float32),
        ],
        compiler_params=pltpu.CompilerParams(needs_layout_passes=False),
    )
    return f(image, depth)


# speedup vs baseline: 1.9190x; 1.2038x over previous
import jax
import jax.numpy as jnp
from jax import lax
from jax.experimental import pallas as pl
from jax.experimental.pallas import tpu as pltpu
from jax.experimental.pallas import tpu_sc as plsc

B, C, H, W = 8, 3, 512, 512
MAX_DISP = 32.0
NC, NS = 2, 16
NW = NC * NS
W_PER_B = NW // B
ROWS_PER_W = H // W_PER_B
RBLK = 16
RH = 8
NBLK = ROWS_PER_W // RBLK
NCH = W // 16
F2P23 = 8388608.0


def _body(image_hbm, depth_hbm, out_l_hbm, out_r_hbm,
          inb, cnt_v, acc_v, in_sem0, in_sem1, out_sem0, out_sem1):
    wid = lax.axis_index("s") * NC + lax.axis_index("c")
    b = wid // W_PER_B
    y_base = (wid % W_PER_B) * ROWS_PER_W
    in_sems = (in_sem0, in_sem1)
    out_sems = (out_sem0, out_sem1)
    out_hbms = (out_l_hbm, out_r_hbm)

    xiota = lax.iota(jnp.int32, 16)
    ones = jnp.ones((16,), jnp.float32)
    zeros = jnp.zeros((16,), jnp.float32)
    dvecs = [jnp.full((16,), di, jnp.int32) for di in range(2)]
    cvecs = [jnp.full((16,), c, jnp.int32) for c in range(C)]

    def issue_in(h, buf):
        ys = y_base + h * RH
        pltpu.async_copy(image_hbm.at[b, :, pl.ds(ys, RH), :],
                         inb.at[buf, pl.ds(0, C)], in_sems[buf])
        pltpu.async_copy(depth_hbm.at[b, pl.ds(ys, RH), :],
                         inb.at[buf, C], in_sems[buf])

    def wait_in(buf):
        pltpu.make_async_copy(image_hbm.at[0, :, pl.ds(0, RH), :],
                              inb.at[buf, pl.ds(0, C)], in_sems[buf]).wait()
        pltpu.make_async_copy(depth_hbm.at[0, pl.ds(0, RH), :],
                              inb.at[buf, C], in_sems[buf]).wait()

    def issue_out(blk, di):
        ys = y_base + blk * RBLK
        pltpu.async_copy(acc_v.at[di],
                         out_hbms[di].at[b, :, pl.ds(ys, RBLK), :],
                         out_sems[di])

    def wait_out(di):
        pltpu.make_async_copy(acc_v.at[di],
                              out_hbms[di].at[0, :, pl.ds(0, RBLK), :],
                              out_sems[di]).wait()

    def zero_dir(di):
        @plsc.parallel_loop(0, RBLK * NCH, 1, unroll=2)
        def zero_k(k):
            r = k // NCH
            xo = (k % NCH) * 16
            cnt_v[di, r, pl.ds(xo, 16)] = zeros
            for c in range(C):
                acc_v[di, c, r, pl.ds(xo, 16)] = zeros

    def scatter_half(buf, roff):
        @plsc.parallel_loop(0, RH * NCH, 1, unroll=2)
        def scat_k(k):
            r = k // NCH
            xo = (k % NCH) * 16
            d16 = inb[buf, C, r, pl.ds(xo, 16)]
            disp = ((d16 * MAX_DISP + F2P23) - F2P23).astype(jnp.int32)
            xb = xiota + xo
            rr = jnp.broadcast_to(r + roff, (16,))
            vals = [inb[buf, c, r, pl.ds(xo, 16)] for c in range(C)]
            for di in range(2):
                xt = xb - disp if di == 0 else xb + disp
                msk = (xt >= 0) & (xt < W)
                xtc = jnp.clip(xt, 0, W - 1)
                plsc.addupdate_scatter(cnt_v, [dvecs[di], rr, xtc],
                                       ones, mask=msk)
                for c in range(C):
                    plsc.addupdate_scatter(acc_v, [dvecs[di], cvecs[c], rr, xtc],
                                           vals[c], mask=msk)

    def fin_dir(di):
        @plsc.parallel_loop(0, RBLK * NCH, 1, unroll=2)
        def fin_k(k):
            r = k // NCH
            xo = (k % NCH) * 16
            cnt = cnt_v[di, r, pl.ds(xo, 16)]
            inv = 1.0 / jnp.maximum(cnt, 1.0)
            for c in range(C):
                a = acc_v[di, c, r, pl.ds(xo, 16)]
                acc_v[di, c, r, pl.ds(xo, 16)] = jnp.clip(a * inv, 0.0, 1.0)

    issue_in(0, 0)

    def do_block(blk, carry):
        issue_in(2 * blk + 1, 1)
        for di in range(2):
            @pl.when(blk > 0)
            def _():
                wait_out(di)
            zero_dir(di)
        wait_in(0)
        scatter_half(0, 0)

        @pl.when(blk < NBLK - 1)
        def _():
            issue_in(2 * blk + 2, 0)
        wait_in(1)
        scatter_half(1, RH)

        for di in range(2):
            fin_dir(di)
            issue_out(blk, di)
        return carry

    lax.fori_loop(0, NBLK, do_block, 0)
    wait_out(0)
    wait_out(1)


def kernel(image, depth):
    mesh = plsc.VectorSubcoreMesh(core_axis_name="c", subcore_axis_name="s",
                                  num_cores=NC, num_subcores=NS)
    f = pl.kernel(
        _body,
        out_type=(jax.ShapeDtypeStruct((B, C, H, W), jnp.float32),
                  jax.ShapeDtypeStruct((B, C, H, W), jnp.float32)),
        mesh=mesh,
        scratch_types=[
            pltpu.VMEM((2, C + 1, RH, W), jnp.float32),
            pltpu.VMEM((2, RBLK, W), jnp.float32),
            pltpu.VMEM((2, C, RBLK, W), jnp.float32),
            pltpu.SemaphoreType.DMA,
            pltpu.SemaphoreType.DMA,
            pltpu.SemaphoreType.DMA,
            pltpu.SemaphoreType.DMA,
        ],
        compiler_params=pltpu.CompilerParams(needs_layout_passes=False),
    )
    return f(image, depth)


# speedup vs baseline: 2.0131x; 1.0490x over previous
import jax
import jax.numpy as jnp
from jax import lax
from jax.experimental import pallas as pl
from jax.experimental.pallas import tpu as pltpu
from jax.experimental.pallas import tpu_sc as plsc

B, C, H, W = 8, 3, 512, 512
MAX_DISP = 32.0
NC, NS = 2, 16
NW = NC * NS
W_PER_B = NW // B
ROWS_PER_W = H // W_PER_B
RBLK = 16
RH = 8
NBLK = ROWS_PER_W // RBLK
NCH = W // 16
F2P23 = 8388608.0


def _body(image_hbm, depth_hbm, out_l_hbm, out_r_hbm,
          inb, cnt_v, acc_v, in_sem0, in_sem1, out_sem0, out_sem1):
    wid = lax.axis_index("s") * NC + lax.axis_index("c")
    b = wid // W_PER_B
    y_base = (wid % W_PER_B) * ROWS_PER_W
    in_sems = (in_sem0, in_sem1)
    out_sems = (out_sem0, out_sem1)
    out_hbms = (out_l_hbm, out_r_hbm)

    xiota = lax.iota(jnp.int32, 16)
    ones = jnp.ones((16,), jnp.float32)
    zeros = jnp.zeros((16,), jnp.float32)
    dvecs = [jnp.full((16,), di, jnp.int32) for di in range(2)]
    cvecs = [jnp.full((16,), c, jnp.int32) for c in range(C)]

    def issue_in(h, buf):
        ys = y_base + h * RH
        pltpu.async_copy(image_hbm.at[b, :, pl.ds(ys, RH), :],
                         inb.at[buf, pl.ds(0, C)], in_sems[buf])
        pltpu.async_copy(depth_hbm.at[b, pl.ds(ys, RH), :],
                         inb.at[buf, C], in_sems[buf])

    def wait_in(buf):
        pltpu.make_async_copy(image_hbm.at[0, :, pl.ds(0, RH), :],
                              inb.at[buf, pl.ds(0, C)], in_sems[buf]).wait()
        pltpu.make_async_copy(depth_hbm.at[0, pl.ds(0, RH), :],
                              inb.at[buf, C], in_sems[buf]).wait()

    def issue_out(blk, di):
        ys = y_base + blk * RBLK
        pltpu.async_copy(acc_v.at[di],
                         out_hbms[di].at[b, :, pl.ds(ys, RBLK), :],
                         out_sems[di])

    def wait_out(di):
        pltpu.make_async_copy(acc_v.at[di],
                              out_hbms[di].at[0, :, pl.ds(0, RBLK), :],
                              out_sems[di]).wait()

    def zero_dir(di):
        @plsc.parallel_loop(0, RBLK * NCH, 1, unroll=4)
        def zero_k(k):
            r = k // NCH
            xo = (k % NCH) * 16
            cnt_v[di, r, pl.ds(xo, 16)] = zeros
            for c in range(C):
                acc_v[di, c, r, pl.ds(xo, 16)] = zeros

    def scatter_half(buf, roff):
        @plsc.parallel_loop(0, RH * NCH, 1, unroll=2)
        def scat_k(k):
            r = k // NCH
            xo = (k % NCH) * 16
            d16 = inb[buf, C, r, pl.ds(xo, 16)]
            disp = ((d16 * MAX_DISP + F2P23) - F2P23).astype(jnp.int32)
            xb = xiota + xo
            rr = jnp.broadcast_to(r + roff, (16,))
            vals = [inb[buf, c, r, pl.ds(xo, 16)] for c in range(C)]
            for di in range(2):
                xt = xb - disp if di == 0 else xb + disp
                msk = (xt >= 0) & (xt < W)
                xtc = jnp.clip(xt, 0, W - 1)
                plsc.addupdate_scatter(cnt_v, [dvecs[di], rr, xtc],
                                       ones, mask=msk)
                for c in range(C):
                    plsc.addupdate_scatter(acc_v, [dvecs[di], cvecs[c], rr, xtc],
                                           vals[c], mask=msk)

    def fin_dir(di):
        @plsc.parallel_loop(0, RBLK * NCH, 1, unroll=4)
        def fin_k(k):
            r = k // NCH
            xo = (k % NCH) * 16
            cnt = cnt_v[di, r, pl.ds(xo, 16)]
            inv = 1.0 / jnp.maximum(cnt, 1.0)
            for c in range(C):
                a = acc_v[di, c, r, pl.ds(xo, 16)]
                acc_v[di, c, r, pl.ds(xo, 16)] = jnp.clip(a * inv, 0.0, 1.0)

    issue_in(0, 0)

    def do_block(blk, carry):
        issue_in(2 * blk + 1, 1)
        for di in range(2):
            @pl.when(blk > 0)
            def _():
                wait_out(di)
            zero_dir(di)
        wait_in(0)
        scatter_half(0, 0)

        @pl.when(blk < NBLK - 1)
        def _():
            issue_in(2 * blk + 2, 0)
        wait_in(1)
        scatter_half(1, RH)

        for di in range(2):
            fin_dir(di)
            issue_out(blk, di)
        return carry

    lax.fori_loop(0, NBLK, do_block, 0)
    wait_out(0)
    wait_out(1)


def kernel(image, depth):
    mesh = plsc.VectorSubcoreMesh(core_axis_name="c", subcore_axis_name="s",
                                  num_cores=NC, num_subcores=NS)
    f = pl.kernel(
        _body,
        out_type=(jax.ShapeDtypeStruct((B, C, H, W), jnp.float32),
                  jax.ShapeDtypeStruct((B, C, H, W), jnp.float32)),
        mesh=mesh,
        scratch_types=[
            pltpu.VMEM((2, C + 1, RH, W), jnp.float32),
            pltpu.VMEM((2, RBLK, W), jnp.float32),
            pltpu.VMEM((2, C, RBLK, W), jnp.float32),
            pltpu.SemaphoreType.DMA,
            pltpu.SemaphoreType.DMA,
            pltpu.SemaphoreType.DMA,
            pltpu.SemaphoreType.DMA,
        ],
        compiler_params=pltpu.CompilerParams(needs_layout_passes=False),
    )
    return f(image, depth)
